# baseline (device time: 77439 ns/iter reference)
import jax
import jax.numpy as jnp
from jax import lax
from jax.experimental import pallas as pl
from jax.experimental.pallas import tpu as pltpu

B, S, D = 2, 512, 2048
H, Dh, Dr = 16, 128, 32
DC = 128
BS = B * S
DP = 256
SCALE = (Dh + Dr) ** -0.5
VMEM_LIMIT = 60 * 1024 * 1024

NJ = 4
DJ = D // NJ
HJ = H // NJ
RJ = HJ * Dr
NI = 2
SI = S // NI


def _exchange_body(x_ref, wdkv_ref, wuk_ref, wuv_ref, wkr_ref,
                   wq_ref, wqr_ref,
                   q_ref, kr_ref, cm_ref, cp_ref,
                   wukm_ref, wukp_ref, wuvm_ref, wuvp_ref,
                   send_sems, recv_sems):
    j = pl.program_id(0)
    my_x = lax.axis_index("x")
    my_y = lax.axis_index("y")
    my_z = lax.axis_index("z")
    peer = (my_x, my_y, 1 - my_z)
    bf16 = jnp.bfloat16

    def _rdma(src, dst, idx):
        return pltpu.make_async_remote_copy(
            src_ref=src, dst_ref=dst,
            send_sem=send_sems.at[idx], recv_sem=recv_sems.at[idx],
            device_id=peer, device_id_type=pl.DeviceIdType.MESH)

    @pl.when(j == 0)
    def _start():
        barrier_sem = pltpu.get_barrier_semaphore()
        pl.semaphore_signal(barrier_sem, inc=1, device_id=peer,
                            device_id_type=pl.DeviceIdType.MESH)
        pl.semaphore_wait(barrier_sem, 1)
        wukm_ref[...] = wuk_ref[...].astype(bf16)
        wuvm_ref[...] = wuv_ref[...].astype(bf16)
        _rdma(wukm_ref, wukp_ref, 0).start()
        _rdma(wuvm_ref, wuvp_ref, 1).start()
        cm_ref[...] = jnp.dot(x_ref[...], wdkv_ref[...],
                              preferred_element_type=jnp.float32).astype(bf16)
        _rdma(cm_ref, cp_ref, 2).start()
        kr_ref[...] = jnp.dot(x_ref[...], wkr_ref[...],
                              preferred_element_type=jnp.float32).astype(bf16)

    q2d = jnp.dot(x_ref[...], wq_ref[...],
                  preferred_element_type=jnp.float32)
    qr2d = jnp.dot(x_ref[...], wqr_ref[...],
                   preferred_element_type=jnp.float32)
    q_ref[...] = jnp.zeros((BS, HJ * DP), bf16)
    for hh in range(HJ):
        q_ref[:, hh * DP:hh * DP + Dh] = (
            q2d[:, hh * Dh:(hh + 1) * Dh].astype(bf16))
        q_ref[:, hh * DP + Dh:hh * DP + Dh + Dr] = (
            qr2d[:, hh * Dr:(hh + 1) * Dr].astype(bf16))

    @pl.when(j == NJ - 1)
    def _finish():
        _rdma(wukm_ref, wukp_ref, 0).wait()
        _rdma(wuvm_ref, wuvp_ref, 1).wait()
        _rdma(cm_ref, cp_ref, 2).wait()


def _attn_body(q_ref, kr_ref, cm_ref, cp_ref,
               wukm_ref, wukp_ref, wuvm_ref, wuvp_ref,
               wo_ref, out_ref, kbig_scr, v_scr, o_scr):
    i = pl.program_id(1)
    bf16 = jnp.bfloat16

    @pl.when(i == 0)
    def _build_kv():
        k2d = (jnp.dot(cm_ref[...], wukm_ref[...],
                       preferred_element_type=jnp.float32)
               + jnp.dot(cp_ref[...], wukp_ref[...],
                         preferred_element_type=jnp.float32))
        v_scr[...] = (jnp.dot(cm_ref[...], wuvm_ref[...],
                              preferred_element_type=jnp.float32)
                      + jnp.dot(cp_ref[...], wuvp_ref[...],
                                preferred_element_type=jnp.float32)
                      ).astype(bf16)
        kbig_scr[...] = jnp.zeros((S, H * DP), bf16)
        kr = kr_ref[...]
        for h in range(H):
            kbig_scr[:, h * DP:h * DP + Dh] = (
                k2d[:, h * Dh:(h + 1) * Dh].astype(bf16))
            kbig_scr[:, h * DP + Dh:h * DP + Dh + Dr] = kr

    for h in range(H):
        q = q_ref[:, h * DP:(h + 1) * DP]
        k = kbig_scr[:, h * DP:(h + 1) * DP]
        v = v_scr[:, h * Dh:(h + 1) * Dh]
        scores = lax.dot_general(q, k, (((1,), (1,)), ((), ())),
                                 preferred_element_type=jnp.float32)
        p = jnp.exp(scores * SCALE)
        o = jnp.dot(p.astype(bf16), v, preferred_element_type=jnp.float32)
        o_scr[:, h * Dh:(h + 1) * Dh] = o / jnp.sum(p, axis=1, keepdims=True)

    out_ref[...] = jnp.dot(o_scr[...], wo_ref[...],
                           preferred_element_type=jnp.float32)


def kernel(x, Wdkv, Wuk, Wuv, Wq, Wqr, Wkr, Wo):
    x2d = x.reshape(BS, D)

    vmem = pl.BlockSpec(memory_space=pltpu.VMEM)
    f32 = jnp.float32
    bf16 = jnp.bfloat16

    whole = lambda shape: pl.BlockSpec(shape, lambda j: tuple(0 for _ in shape))

    qbig, kr2d, cm, cp, wukm, wukp, wuvm, wuvp = pl.pallas_call(
        _exchange_body,
        grid=(NJ,),
        out_shape=(
            jax.ShapeDtypeStruct((BS, H * DP), bf16),
            jax.ShapeDtypeStruct((BS, Dr), bf16),
            jax.ShapeDtypeStruct((BS, DC), bf16),
            jax.ShapeDtypeStruct((BS, DC), bf16),
            jax.ShapeDtypeStruct((DC, D), bf16),
            jax.ShapeDtypeStruct((DC, D), bf16),
            jax.ShapeDtypeStruct((DC, D), bf16),
            jax.ShapeDtypeStruct((DC, D), bf16),
        ),
        in_specs=[
            vmem,
            vmem, vmem, vmem, vmem,
            pl.BlockSpec((D, DJ), lambda j: (0, j)),
            pl.BlockSpec((D, RJ), lambda j: (0, j)),
        ],
        out_specs=(
            pl.BlockSpec((BS, HJ * DP), lambda j: (0, j)),
            whole((BS, Dr)),
            whole((BS, DC)),
            whole((BS, DC)),
            whole((DC, D)),
            whole((DC, D)),
            whole((DC, D)),
            whole((DC, D)),
        ),
        scratch_shapes=[
            pltpu.SemaphoreType.DMA((3,)),
            pltpu.SemaphoreType.DMA((3,)),
        ],
        compiler_params=pltpu.CompilerParams(collective_id=0,
                                             vmem_limit_bytes=VMEM_LIMIT),
    )(x2d, Wdkv, Wuk, Wuv, Wkr, Wq, Wqr)

    out2d = pl.pallas_call(
        _attn_body,
        grid=(B, NI),
        in_specs=[
            pl.BlockSpec((SI, H * DP), lambda b, i: (NI * b + i, 0)),
            pl.BlockSpec((S, Dr), lambda b, i: (b, 0)),
            pl.BlockSpec((S, DC), lambda b, i: (b, 0)),
            pl.BlockSpec((S, DC), lambda b, i: (b, 0)),
            vmem, vmem, vmem, vmem,
            vmem,
        ],
        out_specs=pl.BlockSpec((SI, D), lambda b, i: (NI * b + i, 0)),
        out_shape=jax.ShapeDtypeStruct((BS, D), f32),
        scratch_shapes=[
            pltpu.VMEM((S, H * DP), bf16),
            pltpu.VMEM((S, D), bf16),
            pltpu.VMEM((SI, D), f32),
        ],
        compiler_params=pltpu.CompilerParams(vmem_limit_bytes=VMEM_LIMIT),
    )(qbig, kr2d, cm, cp, wukm, wukp, wuvm, wuvp, Wo)

    return out2d.reshape(B, S, D)


# device time: 66429 ns/iter; 1.1657x vs baseline; 1.1657x over previous
import jax
import jax.numpy as jnp
from jax import lax
from jax.experimental import pallas as pl
from jax.experimental.pallas import tpu as pltpu

B, S, D = 2, 512, 2048
H, Dh, Dr = 16, 128, 32
DC = 128
BS = B * S
DP = 256
SCALE = (Dh + Dr) ** -0.5
VMEM_LIMIT = 60 * 1024 * 1024

NJ = 4
DJ = D // NJ
HJ = H // NJ
RJ = HJ * Dr
NI = 1
SI = S // NI


def _exchange_body(x_ref, wdkv_ref, wuk_ref, wuv_ref, wkr_ref,
                   wq_ref, wqr_ref,
                   q_ref, kr_ref, cfull_ref, wukf_ref, wuvf_ref,
                   c_send, c_recv, wuk_send, wuk_recv, wuv_send, wuv_recv,
                   qr_scr, send_sems, recv_sems):
    j = pl.program_id(0)
    my_x = lax.axis_index("x")
    my_y = lax.axis_index("y")
    my_z = lax.axis_index("z")
    peer = (my_x, my_y, 1 - my_z)
    bf16 = jnp.bfloat16

    def _rdma(src, dst, idx):
        return pltpu.make_async_remote_copy(
            src_ref=src, dst_ref=dst,
            send_sem=send_sems.at[idx], recv_sem=recv_sems.at[idx],
            device_id=peer, device_id_type=pl.DeviceIdType.MESH)

    @pl.when(j == 0)
    def _start():
        barrier_sem = pltpu.get_barrier_semaphore()
        pl.semaphore_signal(barrier_sem, inc=1, device_id=peer,
                            device_id_type=pl.DeviceIdType.MESH)
        pl.semaphore_wait(barrier_sem, 1)
        wuk_send[...] = wuk_ref[...].astype(bf16)
        wuv_send[...] = wuv_ref[...].astype(bf16)
        _rdma(wuk_send, wuk_recv, 0).start()
        _rdma(wuv_send, wuv_recv, 1).start()
        c_send[...] = jnp.dot(x_ref[...], wdkv_ref[...],
                              preferred_element_type=jnp.float32).astype(bf16)
        _rdma(c_send, c_recv, 2).start()
        kr_ref[...] = jnp.dot(x_ref[...], wkr_ref[...],
                              preferred_element_type=jnp.float32).astype(bf16)
        qr_all = jnp.dot(x_ref[...], wqr_ref[...],
                         preferred_element_type=jnp.float32)
        for jj in range(NJ):
            qr_scr[jj] = qr_all[:, jj * RJ:(jj + 1) * RJ].astype(bf16)

    q2d = jnp.dot(x_ref[...], wq_ref[...],
                  preferred_element_type=jnp.float32)
    qr2d = qr_scr[j]
    q_ref[...] = jnp.zeros((BS, HJ * DP), bf16)
    for hh in range(HJ):
        q_ref[:, hh * DP:hh * DP + Dh] = (
            q2d[:, hh * Dh:(hh + 1) * Dh].astype(bf16))
        q_ref[:, hh * DP + Dh:hh * DP + Dh + Dr] = (
            qr2d[:, hh * Dr:(hh + 1) * Dr])

    @pl.when(j == NJ - 1)
    def _finish():
        _rdma(wuk_send, wuk_recv, 0).wait()
        _rdma(wuv_send, wuv_recv, 1).wait()
        _rdma(c_send, c_recv, 2).wait()
        cfull_ref[:, pl.ds(my_z * DC, DC)] = c_send[...]
        cfull_ref[:, pl.ds((1 - my_z) * DC, DC)] = c_recv[...]
        wukf_ref[pl.ds(my_z * DC, DC), :] = wuk_send[...]
        wukf_ref[pl.ds((1 - my_z) * DC, DC), :] = wuk_recv[...]
        wuvf_ref[pl.ds(my_z * DC, DC), :] = wuv_send[...]
        wuvf_ref[pl.ds((1 - my_z) * DC, DC), :] = wuv_recv[...]


def _attn_body(q_ref, kr_ref, cfull_ref, wukf_ref, wuvf_ref,
               wo_ref, out_ref, kbig_scr, v_scr, o_scr):
    i = pl.program_id(1)
    bf16 = jnp.bfloat16

    @pl.when(i == 0)
    def _build_kv():
        k2d = jnp.dot(cfull_ref[...], wukf_ref[...],
                      preferred_element_type=jnp.float32)
        v_scr[...] = jnp.dot(cfull_ref[...], wuvf_ref[...],
                             preferred_element_type=jnp.float32).astype(bf16)
        kbig_scr[...] = jnp.zeros((S, H * DP), bf16)
        kr = kr_ref[...]
        for h in range(H):
            kbig_scr[:, h * DP:h * DP + Dh] = (
                k2d[:, h * Dh:(h + 1) * Dh].astype(bf16))
            kbig_scr[:, h * DP + Dh:h * DP + Dh + Dr] = kr

    for h in range(H):
        q = q_ref[:, h * DP:(h + 1) * DP]
        k = kbig_scr[:, h * DP:(h + 1) * DP]
        v = v_scr[:, h * Dh:(h + 1) * Dh]
        scores = lax.dot_general(q, k, (((1,), (1,)), ((), ())),
                                 preferred_element_type=jnp.float32)
        p = jnp.exp(scores * SCALE)
        o = jnp.dot(p, v, preferred_element_type=jnp.float32)
        o_scr[:, h * Dh:(h + 1) * Dh] = o / jnp.sum(p, axis=1, keepdims=True)

    out_ref[...] = jnp.dot(o_scr[...], wo_ref[...],
                           preferred_element_type=jnp.float32)


def kernel(x, Wdkv, Wuk, Wuv, Wq, Wqr, Wkr, Wo):
    x2d = x.reshape(BS, D)

    vmem = pl.BlockSpec(memory_space=pltpu.VMEM)
    f32 = jnp.float32
    bf16 = jnp.bfloat16

    qbig, kr2d, cfull, wukf, wuvf = pl.pallas_call(
        _exchange_body,
        grid=(NJ,),
        out_shape=(
            jax.ShapeDtypeStruct((BS, H * DP), bf16),
            jax.ShapeDtypeStruct((BS, Dr), bf16),
            jax.ShapeDtypeStruct((BS, 2 * DC), bf16),
            jax.ShapeDtypeStruct((2 * DC, D), bf16),
            jax.ShapeDtypeStruct((2 * DC, D), bf16),
        ),
        in_specs=[
            vmem,
            vmem, vmem, vmem, vmem,
            pl.BlockSpec((D, DJ), lambda j: (0, j)),
            vmem,
        ],
        out_specs=(
            pl.BlockSpec((BS, HJ * DP), lambda j: (0, j)),
            pl.BlockSpec((BS, Dr), lambda j: (0, 0)),
            pl.BlockSpec((BS, 2 * DC), lambda j: (0, 0)),
            pl.BlockSpec((2 * DC, D), lambda j: (0, 0)),
            pl.BlockSpec((2 * DC, D), lambda j: (0, 0)),
        ),
        scratch_shapes=[
            pltpu.VMEM((BS, DC), bf16),
            pltpu.VMEM((BS, DC), bf16),
            pltpu.VMEM((DC, D), bf16),
            pltpu.VMEM((DC, D), bf16),
            pltpu.VMEM((DC, D), bf16),
            pltpu.VMEM((DC, D), bf16),
            pltpu.VMEM((NJ, BS, RJ), bf16),
            pltpu.SemaphoreType.DMA((3,)),
            pltpu.SemaphoreType.DMA((3,)),
        ],
        compiler_params=pltpu.CompilerParams(collective_id=0,
                                             vmem_limit_bytes=VMEM_LIMIT),
    )(x2d, Wdkv, Wuk, Wuv, Wkr, Wq, Wqr)

    out2d = pl.pallas_call(
        _attn_body,
        grid=(B, NI),
        in_specs=[
            pl.BlockSpec((SI, H * DP), lambda b, i: (NI * b + i, 0)),
            pl.BlockSpec((S, Dr), lambda b, i: (b, 0)),
            pl.BlockSpec((S, 2 * DC), lambda b, i: (b, 0)),
            vmem, vmem,
            vmem,
        ],
        out_specs=pl.BlockSpec((SI, D), lambda b, i: (NI * b + i, 0)),
        out_shape=jax.ShapeDtypeStruct((BS, D), f32),
        scratch_shapes=[
            pltpu.VMEM((S, H * DP), bf16),
            pltpu.VMEM((S, D), bf16),
            pltpu.VMEM((SI, D), f32),
        ],
        compiler_params=pltpu.CompilerParams(vmem_limit_bytes=VMEM_LIMIT),
    )(qbig, kr2d, cfull, wukf, wuvf, Wo)

    return out2d.reshape(B, S, D)
